# Initial kernel scaffold; baseline (speedup 1.0000x reference)
import jax
import jax.numpy as jnp
from jax.experimental import pallas as pl


def kernel(X, Wg, W1, W2, W3):
    raise NotImplementedError("write your pallas kernel here")


# speedup vs baseline: 1.0712x; 1.0712x over previous
import functools

import jax
import jax.numpy as jnp
from jax.experimental import pallas as pl
from jax.experimental.pallas import tpu as pltpu

_D = 768
_E = 8
_H = 3072
_T = 2048
_TT = 512


def _silu(x):
    return x / (1.0 + jnp.exp(-x))


def _moe_body(x_ref, wg_ref, w1_ref, w3_ref, w2_ref, out_ref):
    e = pl.program_id(0)
    t = pl.program_id(1)
    x = x_ref[...]
    logits = jnp.dot(x, wg_ref[...], preferred_element_type=jnp.float32)
    iota = jax.lax.broadcasted_iota(jnp.int32, logits.shape, 1)
    m1 = jnp.max(logits, axis=1, keepdims=True)
    idx1 = jnp.min(jnp.where(logits == m1, iota, _E), axis=1, keepdims=True)
    neg = jnp.float32(-3.0e38)
    logits2 = jnp.where(iota == idx1, neg, logits)
    m2 = jnp.max(logits2, axis=1, keepdims=True)
    idx2 = jnp.min(jnp.where(logits2 == m2, iota, _E), axis=1, keepdims=True)
    z = jnp.exp(m2 - m1)
    g1 = 1.0 / (1.0 + z)
    g2 = z / (1.0 + z)
    combine = jnp.where(idx1 == e, g1, jnp.where(idx2 == e, g2, 0.0))
    xs = (x * combine).astype(jnp.bfloat16)
    h1 = jnp.dot(xs, w1_ref[0], preferred_element_type=jnp.float32)
    h3 = jnp.dot(xs, w3_ref[0], preferred_element_type=jnp.float32)
    hh = (_silu(h1) * h3).astype(jnp.bfloat16)
    y = jnp.dot(hh, w2_ref[0], preferred_element_type=jnp.float32)

    rows = pl.ds(t * _TT, _TT)

    @pl.when(e == 0)
    def _():
        out_ref[rows, :] = x + y

    @pl.when(e != 0)
    def _():
        out_ref[rows, :] = out_ref[rows, :] + y


def kernel(X, Wg, W1, W2, W3):
    x2 = X.reshape(_T, _D)
    w1 = W1.astype(jnp.bfloat16)
    w3 = W3.astype(jnp.bfloat16)
    w2 = W2.astype(jnp.bfloat16)
    nt = _T // _TT
    out = pl.pallas_call(
        _moe_body,
        grid=(_E, nt),
        in_specs=[
            pl.BlockSpec((_TT, _D), lambda e, t: (t, 0)),
            pl.BlockSpec((_D, _E), lambda e, t: (0, 0)),
            pl.BlockSpec((1, _D, _H), lambda e, t: (e, 0, 0)),
            pl.BlockSpec((1, _D, _H), lambda e, t: (e, 0, 0)),
            pl.BlockSpec((1, _H, _D), lambda e, t: (e, 0, 0)),
        ],
        out_specs=pl.BlockSpec((_T, _D), lambda e, t: (0, 0)),
        out_shape=jax.ShapeDtypeStruct((_T, _D), jnp.float32),
        compiler_params=pltpu.CompilerParams(
            dimension_semantics=("arbitrary", "arbitrary"),
        ),
    )(x2, Wg, w1, w3, w2)
    return out.reshape(X.shape)


# speedup vs baseline: 1.2587x; 1.1750x over previous
import functools

import jax
import jax.numpy as jnp
from jax import lax
from jax.experimental import pallas as pl
from jax.experimental.pallas import tpu as pltpu
from jax.experimental.pallas import tpu_sc as plsc

_D = 768
_E = 8
_H = 3072
_T = 2048
_S = 2
_R = _T * _S
_TT = 256
_NT = _R // _TT

_NW = 32
_CHUNK = _T // _NW


def _silu(x):
    return x / (1.0 + jnp.exp(-x))


def _routing_body(x_ref, wg_ref, xg0_ref, xg1_ref, pos0_ref, pos1_ref,
                  offs_ref):
    x = x_ref[...]
    logits = jnp.dot(x, wg_ref[...], preferred_element_type=jnp.float32)
    iota = lax.broadcasted_iota(jnp.int32, logits.shape, 1)
    m1 = jnp.max(logits, axis=1, keepdims=True)
    oh1 = logits == m1
    idx1 = jnp.min(jnp.where(oh1, iota, _E), axis=1, keepdims=True)
    oh1 = iota == idx1
    neg = jnp.float32(-3.0e38)
    logits2 = jnp.where(oh1, neg, logits)
    m2 = jnp.max(logits2, axis=1, keepdims=True)
    idx2 = jnp.min(jnp.where(logits2 == m2, iota, _E), axis=1, keepdims=True)
    oh2 = iota == idx2
    z = jnp.exp(m2 - m1)
    g1 = 1.0 / (1.0 + z)
    g2 = z / (1.0 + z)

    m = oh1.astype(jnp.float32) + oh2.astype(jnp.float32)
    r_iota = lax.broadcasted_iota(jnp.int32, (_T, _T), 0)
    c_iota = lax.broadcasted_iota(jnp.int32, (_T, _T), 1)
    tril = (r_iota > c_iota).astype(jnp.float32)
    rank = jnp.dot(tril, m, preferred_element_type=jnp.float32)
    counts = jnp.sum(m, axis=0, keepdims=True)
    c16 = jnp.concatenate([counts, jnp.zeros((1, 16 - _E), jnp.float32)],
                          axis=1)
    s = jnp.concatenate([jnp.zeros((1, 1), jnp.float32), c16[:, :15]],
                        axis=1)
    for k in (1, 2, 4, 8):
        s = s + jnp.concatenate(
            [jnp.zeros((1, k), jnp.float32), s[:, :16 - k]], axis=1)
    offs = s

    off8 = offs[:, :_E]
    o1 = jnp.sum(jnp.where(oh1, off8, 0.0), axis=1, keepdims=True)
    o2 = jnp.sum(jnp.where(oh2, off8, 0.0), axis=1, keepdims=True)
    r1 = jnp.sum(jnp.where(oh1, rank, 0.0), axis=1, keepdims=True)
    r2 = jnp.sum(jnp.where(oh2, rank, 0.0), axis=1, keepdims=True)
    pos0_ref[...] = (o1 + r1).astype(jnp.int32)
    pos1_ref[...] = (o2 + r2).astype(jnp.int32)
    xg0_ref[...] = x * g1
    xg1_ref[...] = x * g2
    offs_ref[...] = offs.astype(jnp.int32)


def _routing(x2, wg):
    return pl.pallas_call(
        _routing_body,
        out_shape=(
            jax.ShapeDtypeStruct((_T, _D), jnp.float32),
            jax.ShapeDtypeStruct((_T, _D), jnp.float32),
            jax.ShapeDtypeStruct((_T, 1), jnp.int32),
            jax.ShapeDtypeStruct((_T, 1), jnp.int32),
            jax.ShapeDtypeStruct((1, 16), jnp.int32),
        ),
    )(x2, wg)


def _sc_mesh():
    return plsc.VectorSubcoreMesh(core_axis_name="c", subcore_axis_name="s")


def _dispatch_body(xg0, xg1, pos0, pos1, xs_out, idx_v, rows_v, sem):
    wid = lax.axis_index("s") * 2 + lax.axis_index("c")
    base = wid * _CHUNK
    pltpu.sync_copy(pos0.at[pl.ds(base, _CHUNK)], idx_v)
    pltpu.sync_copy(xg0.at[pl.ds(base, _CHUNK)], rows_v)
    pltpu.async_copy(rows_v, xs_out.at[idx_v], sem).wait()
    pltpu.sync_copy(pos1.at[pl.ds(base, _CHUNK)], idx_v)
    pltpu.sync_copy(xg1.at[pl.ds(base, _CHUNK)], rows_v)
    pltpu.async_copy(rows_v, xs_out.at[idx_v], sem).wait()


def _dispatch(xg0, xg1, pos0, pos1):
    k = functools.partial(
        pl.kernel,
        mesh=_sc_mesh(),
        out_type=jax.ShapeDtypeStruct((_R, _D), jnp.float32),
        scratch_types=[
            pltpu.VMEM((_CHUNK,), jnp.int32),
            pltpu.VMEM((_CHUNK, _D), jnp.float32),
            pltpu.SemaphoreType.DMA,
        ],
    )(_dispatch_body)
    return k(xg0, xg1, pos0, pos1)


_SUB = 32


def _combine_body(x, y, pos0, pos1, out, idx_v, a_v, b_v, sem):
    wid = lax.axis_index("s") * 2 + lax.axis_index("c")
    base = wid * _CHUNK

    def addinto(i, carry):
        def col(c, carry2):
            sl = pl.ds(c * 16, 16)
            a_v[i, sl] = a_v[i, sl] + b_v[i, sl]
            return carry2

        lax.fori_loop(0, _D // 16, col, 0, unroll=4)
        return carry

    for s in range(_CHUNK // _SUB):
        b2 = base + s * _SUB
        pltpu.sync_copy(pos0.at[pl.ds(b2, _SUB)], idx_v)
        pltpu.async_copy(y.at[idx_v], a_v, sem).wait()
        pltpu.sync_copy(pos1.at[pl.ds(b2, _SUB)], idx_v)
        pltpu.async_copy(y.at[idx_v], b_v, sem).wait()
        lax.fori_loop(0, _SUB, addinto, 0)
        pltpu.sync_copy(x.at[pl.ds(b2, _SUB)], b_v)
        lax.fori_loop(0, _SUB, addinto, 0)
        pltpu.sync_copy(a_v, out.at[pl.ds(b2, _SUB)])


def _combine(x2, y, pos0, pos1):
    k = functools.partial(
        pl.kernel,
        mesh=_sc_mesh(),
        out_type=jax.ShapeDtypeStruct((_T, _D), jnp.float32),
        scratch_types=[
            pltpu.VMEM((_SUB,), jnp.int32),
            pltpu.VMEM((_SUB, _D), jnp.float32),
            pltpu.VMEM((_SUB, _D), jnp.float32),
            pltpu.SemaphoreType.DMA,
        ],
    )(_combine_body)
    return k(x2, y, pos0, pos1)


def _ffn_body(offs_ref, xs_ref, w1_ref, w3_ref, w2_ref, out_ref):
    e = pl.program_id(0)
    t = pl.program_id(1)

    @pl.when(jnp.logical_and(e == 0, t == 0))
    def _():
        out_ref[...] = jnp.zeros_like(out_ref)

    start = jnp.maximum(offs_ref[0, e], t * _TT)
    end = jnp.minimum(offs_ref[0, e + 1], (t + 1) * _TT)

    @pl.when(start < end)
    def _():
        rows = pl.ds(t * _TT, _TT)
        x = xs_ref[rows, :]
        riota = t * _TT + lax.broadcasted_iota(jnp.int32, (_TT, 1), 0)
        valid = (riota >= start) & (riota < end)
        xb = jnp.where(valid, x, 0.0).astype(jnp.bfloat16)
        h1 = jnp.dot(xb, w1_ref[0], preferred_element_type=jnp.float32)
        h3 = jnp.dot(xb, w3_ref[0], preferred_element_type=jnp.float32)
        hh = (_silu(h1) * h3).astype(jnp.bfloat16)
        y = jnp.dot(hh, w2_ref[0], preferred_element_type=jnp.float32)
        out_ref[rows, :] += y


def _grouped_ffn(offs, xs, w1, w3, w2):
    grid_spec = pltpu.PrefetchScalarGridSpec(
        num_scalar_prefetch=1,
        grid=(_E, _NT),
        in_specs=[
            pl.BlockSpec((_R, _D), lambda e, t, o: (0, 0)),
            pl.BlockSpec((1, _D, _H), lambda e, t, o: (e, 0, 0)),
            pl.BlockSpec((1, _D, _H), lambda e, t, o: (e, 0, 0)),
            pl.BlockSpec((1, _H, _D), lambda e, t, o: (e, 0, 0)),
        ],
        out_specs=pl.BlockSpec((_R, _D), lambda e, t, o: (0, 0)),
    )
    return pl.pallas_call(
        _ffn_body,
        grid_spec=grid_spec,
        out_shape=jax.ShapeDtypeStruct((_R, _D), jnp.float32),
        compiler_params=pltpu.CompilerParams(
            dimension_semantics=("arbitrary", "arbitrary"),
        ),
    )(offs, xs, w1, w3, w2)


def kernel(X, Wg, W1, W2, W3):
    x2 = X.reshape(_T, _D)
    w1 = W1.astype(jnp.bfloat16)
    w3 = W3.astype(jnp.bfloat16)
    w2 = W2.astype(jnp.bfloat16)

    xg0, xg1, pos0, pos1, offs = _routing(x2, Wg)
    p0 = pos0.reshape(_R // 2)
    p1 = pos1.reshape(_R // 2)

    xs = _dispatch(xg0, xg1, p0, p1)
    y = _grouped_ffn(offs, xs, w1, w3, w2)
    out = _combine(x2, y, p0, p1)
    return out.reshape(X.shape)


# speedup vs baseline: 1.2852x; 1.0211x over previous
import functools

import jax
import jax.numpy as jnp
from jax import lax
from jax.experimental import pallas as pl
from jax.experimental.pallas import tpu as pltpu
from jax.experimental.pallas import tpu_sc as plsc

_D = 768
_E = 8
_H = 3072
_T = 2048
_S = 2
_R = _T * _S
_TT = 256
_NT = _R // _TT

_NW = 32
_CHUNK = _T // _NW


def _silu(x):
    return x / (1.0 + jnp.exp(-x))


def _routing_body(x_ref, wg_ref, xg0_ref, xg1_ref, pos0_ref, pos1_ref,
                  offs_ref):
    x = x_ref[...]
    logits = jnp.dot(x, wg_ref[...], preferred_element_type=jnp.float32)
    iota = lax.broadcasted_iota(jnp.int32, logits.shape, 1)
    m1 = jnp.max(logits, axis=1, keepdims=True)
    oh1 = logits == m1
    idx1 = jnp.min(jnp.where(oh1, iota, _E), axis=1, keepdims=True)
    oh1 = iota == idx1
    neg = jnp.float32(-3.0e38)
    logits2 = jnp.where(oh1, neg, logits)
    m2 = jnp.max(logits2, axis=1, keepdims=True)
    idx2 = jnp.min(jnp.where(logits2 == m2, iota, _E), axis=1, keepdims=True)
    oh2 = iota == idx2
    z = jnp.exp(m2 - m1)
    g1 = 1.0 / (1.0 + z)
    g2 = z / (1.0 + z)

    m = oh1.astype(jnp.float32) + oh2.astype(jnp.float32)
    r_iota = lax.broadcasted_iota(jnp.int32, (_T, _T), 0)
    c_iota = lax.broadcasted_iota(jnp.int32, (_T, _T), 1)
    tril = (r_iota > c_iota).astype(jnp.float32)
    rank = jnp.dot(tril, m, preferred_element_type=jnp.float32)
    counts = jnp.sum(m, axis=0, keepdims=True)
    c16 = jnp.concatenate([counts, jnp.zeros((1, 16 - _E), jnp.float32)],
                          axis=1)
    s = jnp.concatenate([jnp.zeros((1, 1), jnp.float32), c16[:, :15]],
                        axis=1)
    for k in (1, 2, 4, 8):
        s = s + jnp.concatenate(
            [jnp.zeros((1, k), jnp.float32), s[:, :16 - k]], axis=1)
    offs = s

    off8 = offs[:, :_E]
    o1 = jnp.sum(jnp.where(oh1, off8, 0.0), axis=1, keepdims=True)
    o2 = jnp.sum(jnp.where(oh2, off8, 0.0), axis=1, keepdims=True)
    r1 = jnp.sum(jnp.where(oh1, rank, 0.0), axis=1, keepdims=True)
    r2 = jnp.sum(jnp.where(oh2, rank, 0.0), axis=1, keepdims=True)
    pos0_ref[...] = (o1 + r1).astype(jnp.int32)
    pos1_ref[...] = (o2 + r2).astype(jnp.int32)
    xg0_ref[...] = x * g1
    xg1_ref[...] = x * g2
    offs_ref[...] = offs.astype(jnp.int32)


def _routing(x2, wg):
    return pl.pallas_call(
        _routing_body,
        out_shape=(
            jax.ShapeDtypeStruct((_T, _D), jnp.float32),
            jax.ShapeDtypeStruct((_T, _D), jnp.float32),
            jax.ShapeDtypeStruct((_T, 1), jnp.int32),
            jax.ShapeDtypeStruct((_T, 1), jnp.int32),
            jax.ShapeDtypeStruct((1, 16), jnp.int32),
        ),
    )(x2, wg)


def _sc_mesh():
    return plsc.VectorSubcoreMesh(core_axis_name="c", subcore_axis_name="s")


def _dispatch_body(xg0, xg1, pos0, pos1, xs_out, idx_v, rows_v, sem):
    wid = lax.axis_index("s") * 2 + lax.axis_index("c")
    base = wid * _CHUNK
    pltpu.sync_copy(pos0.at[pl.ds(base, _CHUNK)], idx_v)
    pltpu.sync_copy(xg0.at[pl.ds(base, _CHUNK)], rows_v)
    pltpu.async_copy(rows_v, xs_out.at[idx_v], sem).wait()
    pltpu.sync_copy(pos1.at[pl.ds(base, _CHUNK)], idx_v)
    pltpu.sync_copy(xg1.at[pl.ds(base, _CHUNK)], rows_v)
    pltpu.async_copy(rows_v, xs_out.at[idx_v], sem).wait()


def _dispatch(xg0, xg1, pos0, pos1):
    k = functools.partial(
        pl.kernel,
        mesh=_sc_mesh(),
        out_type=jax.ShapeDtypeStruct((_R, _D), jnp.float32),
        scratch_types=[
            pltpu.VMEM((_CHUNK,), jnp.int32),
            pltpu.VMEM((_CHUNK, _D), jnp.float32),
            pltpu.SemaphoreType.DMA,
        ],
    )(_dispatch_body)
    return k(xg0, xg1, pos0, pos1)


_SUB = 32


def _combine_body(x, y, pos0, pos1, out, idx_v, a_v, b_v, sem):
    wid = lax.axis_index("s") * 2 + lax.axis_index("c")
    base = wid * _CHUNK

    def addinto(i, carry):
        def col(c, carry2):
            sl = pl.ds(c * 16, 16)
            a_v[i, sl] = a_v[i, sl] + b_v[i, sl]
            return carry2

        lax.fori_loop(0, _D // 16, col, 0, unroll=4)
        return carry

    for s in range(_CHUNK // _SUB):
        b2 = base + s * _SUB
        pltpu.sync_copy(pos0.at[pl.ds(b2, _SUB)], idx_v)
        pltpu.async_copy(y.at[idx_v], a_v, sem).wait()
        pltpu.sync_copy(pos1.at[pl.ds(b2, _SUB)], idx_v)
        pltpu.async_copy(y.at[idx_v], b_v, sem).wait()
        lax.fori_loop(0, _SUB, addinto, 0)
        pltpu.sync_copy(x.at[pl.ds(b2, _SUB)], b_v)
        lax.fori_loop(0, _SUB, addinto, 0)
        pltpu.sync_copy(a_v, out.at[pl.ds(b2, _SUB)])


def _combine(x2, y, pos0, pos1):
    k = functools.partial(
        pl.kernel,
        mesh=_sc_mesh(),
        out_type=jax.ShapeDtypeStruct((_T, _D), jnp.float32),
        scratch_types=[
            pltpu.VMEM((_SUB,), jnp.int32),
            pltpu.VMEM((_SUB, _D), jnp.float32),
            pltpu.VMEM((_SUB, _D), jnp.float32),
            pltpu.SemaphoreType.DMA,
        ],
    )(_combine_body)
    return k(x2, y, pos0, pos1)


_HB = 1024
_NH = _H // _HB


def _ffn_body(offs_ref, xs_ref, w1_ref, w3_ref, w2_ref, out_ref,
              w1c, w3c, w2c):
    e = pl.program_id(0)
    h = pl.program_id(1)
    t = pl.program_id(2)

    @pl.when(jnp.logical_and(e == 0, jnp.logical_and(h == 0, t == 0)))
    def _():
        out_ref[...] = jnp.zeros_like(out_ref)

    @pl.when(t == 0)
    def _():
        w1c[...] = w1_ref[0].astype(jnp.bfloat16)
        w3c[...] = w3_ref[0].astype(jnp.bfloat16)
        w2c[...] = w2_ref[0].astype(jnp.bfloat16)

    start = jnp.maximum(offs_ref[0, e], t * _TT)
    end = jnp.minimum(offs_ref[0, e + 1], (t + 1) * _TT)

    @pl.when(start < end)
    def _():
        rows = pl.ds(t * _TT, _TT)
        x = xs_ref[rows, :]
        riota = t * _TT + lax.broadcasted_iota(jnp.int32, (_TT, 1), 0)
        valid = (riota >= start) & (riota < end)
        xb = jnp.where(valid, x, 0.0).astype(jnp.bfloat16)
        h1 = jnp.dot(xb, w1c[...], preferred_element_type=jnp.float32)
        h3 = jnp.dot(xb, w3c[...], preferred_element_type=jnp.float32)
        hh = (_silu(h1) * h3).astype(jnp.bfloat16)
        y = jnp.dot(hh, w2c[...], preferred_element_type=jnp.float32)
        out_ref[rows, :] += y


def _grouped_ffn(offs, xs, w1, w3, w2):
    grid_spec = pltpu.PrefetchScalarGridSpec(
        num_scalar_prefetch=1,
        grid=(_E, _NH, _NT),
        in_specs=[
            pl.BlockSpec((_R, _D), lambda e, h, t, o: (0, 0)),
            pl.BlockSpec((1, _D, _HB), lambda e, h, t, o: (e, 0, h)),
            pl.BlockSpec((1, _D, _HB), lambda e, h, t, o: (e, 0, h)),
            pl.BlockSpec((1, _HB, _D), lambda e, h, t, o: (e, h, 0)),
        ],
        out_specs=pl.BlockSpec((_R, _D), lambda e, h, t, o: (0, 0)),
        scratch_shapes=[
            pltpu.VMEM((_D, _HB), jnp.bfloat16),
            pltpu.VMEM((_D, _HB), jnp.bfloat16),
            pltpu.VMEM((_HB, _D), jnp.bfloat16),
        ],
    )
    return pl.pallas_call(
        _ffn_body,
        grid_spec=grid_spec,
        out_shape=jax.ShapeDtypeStruct((_R, _D), jnp.float32),
        compiler_params=pltpu.CompilerParams(
            dimension_semantics=("arbitrary", "arbitrary", "arbitrary"),
        ),
    )(offs, xs, w1, w3, w2)


def kernel(X, Wg, W1, W2, W3):
    x2 = X.reshape(_T, _D)

    xg0, xg1, pos0, pos1, offs = _routing(x2, Wg)
    p0 = pos0.reshape(_R // 2)
    p1 = pos1.reshape(_R // 2)

    xs = _dispatch(xg0, xg1, p0, p1)
    y = _grouped_ffn(offs, xs, W1, W3, W2)
    out = _combine(x2, y, p0, p1)
    return out.reshape(X.shape)


# speedup vs baseline: 1.7479x; 1.3601x over previous
import functools

import jax
import jax.numpy as jnp
from jax import lax
from jax.experimental import pallas as pl
from jax.experimental.pallas import tpu as pltpu
from jax.experimental.pallas import tpu_sc as plsc

_D = 768
_E = 8
_H = 3072
_T = 2048
_S = 2
_R = _T * _S
_TT = 256
_NT = _R // _TT

_NW = 32
_CHUNK = _T // _NW


def _silu(x):
    return x / (1.0 + jnp.exp(-x))


def _routing_body(x_ref, wg_ref, xg0_ref, xg1_ref, pos0_ref, pos1_ref,
                  offs_ref):
    x = x_ref[...]
    logits = jnp.dot(x, wg_ref[...], preferred_element_type=jnp.float32)
    iota = lax.broadcasted_iota(jnp.int32, logits.shape, 1)
    m1 = jnp.max(logits, axis=1, keepdims=True)
    oh1 = logits == m1
    idx1 = jnp.min(jnp.where(oh1, iota, _E), axis=1, keepdims=True)
    oh1 = iota == idx1
    neg = jnp.float32(-3.0e38)
    logits2 = jnp.where(oh1, neg, logits)
    m2 = jnp.max(logits2, axis=1, keepdims=True)
    idx2 = jnp.min(jnp.where(logits2 == m2, iota, _E), axis=1, keepdims=True)
    oh2 = iota == idx2
    z = jnp.exp(m2 - m1)
    g1 = 1.0 / (1.0 + z)
    g2 = z / (1.0 + z)

    m = oh1.astype(jnp.float32) + oh2.astype(jnp.float32)
    r_iota = lax.broadcasted_iota(jnp.int32, (_T, _T), 0)
    c_iota = lax.broadcasted_iota(jnp.int32, (_T, _T), 1)
    tril = (r_iota > c_iota).astype(jnp.float32)
    rank = jnp.dot(tril, m, preferred_element_type=jnp.float32)
    counts = jnp.sum(m, axis=0, keepdims=True)
    c16 = jnp.concatenate([counts, jnp.zeros((1, 16 - _E), jnp.float32)],
                          axis=1)
    s = jnp.concatenate([jnp.zeros((1, 1), jnp.float32), c16[:, :15]],
                        axis=1)
    for k in (1, 2, 4, 8):
        s = s + jnp.concatenate(
            [jnp.zeros((1, k), jnp.float32), s[:, :16 - k]], axis=1)
    offs = s

    off8 = offs[:, :_E]
    o1 = jnp.sum(jnp.where(oh1, off8, 0.0), axis=1, keepdims=True)
    o2 = jnp.sum(jnp.where(oh2, off8, 0.0), axis=1, keepdims=True)
    r1 = jnp.sum(jnp.where(oh1, rank, 0.0), axis=1, keepdims=True)
    r2 = jnp.sum(jnp.where(oh2, rank, 0.0), axis=1, keepdims=True)
    pos0_ref[...] = (o1 + r1).astype(jnp.int32)
    pos1_ref[...] = (o2 + r2).astype(jnp.int32)
    xg0_ref[...] = x * g1
    xg1_ref[...] = x * g2
    offs_ref[...] = offs.astype(jnp.int32)


def _routing(x2, wg):
    return pl.pallas_call(
        _routing_body,
        out_shape=(
            jax.ShapeDtypeStruct((_T, _D), jnp.float32),
            jax.ShapeDtypeStruct((_T, _D), jnp.float32),
            jax.ShapeDtypeStruct((_T, 1), jnp.int32),
            jax.ShapeDtypeStruct((_T, 1), jnp.int32),
            jax.ShapeDtypeStruct((1, 16), jnp.int32),
        ),
    )(x2, wg)


def _sc_mesh():
    return plsc.VectorSubcoreMesh(core_axis_name="c", subcore_axis_name="s")


def _dispatch_body(xg0, xg1, pos0, pos1, xs_out, idx_v, rows_v, sem):
    wid = lax.axis_index("s") * 2 + lax.axis_index("c")
    base = wid * _CHUNK
    pltpu.sync_copy(pos0.at[pl.ds(base, _CHUNK)], idx_v)
    pltpu.sync_copy(xg0.at[pl.ds(base, _CHUNK)], rows_v)
    pltpu.async_copy(rows_v, xs_out.at[idx_v], sem).wait()
    pltpu.sync_copy(pos1.at[pl.ds(base, _CHUNK)], idx_v)
    pltpu.sync_copy(xg1.at[pl.ds(base, _CHUNK)], rows_v)
    pltpu.async_copy(rows_v, xs_out.at[idx_v], sem).wait()


def _dispatch(xg0, xg1, pos0, pos1):
    k = functools.partial(
        pl.kernel,
        mesh=_sc_mesh(),
        out_type=jax.ShapeDtypeStruct((_R, _D), jnp.float32),
        scratch_types=[
            pltpu.VMEM((_CHUNK,), jnp.int32),
            pltpu.VMEM((_CHUNK, _D), jnp.float32),
            pltpu.SemaphoreType.DMA,
        ],
    )(_dispatch_body)
    return k(xg0, xg1, pos0, pos1)


_SUB = 32


def _combine_body(x, y, pos0, pos1, out, idx_v, a_v, b_v, sem):
    wid = lax.axis_index("s") * 2 + lax.axis_index("c")
    base = wid * _CHUNK

    def addinto(i, carry):
        def col(c, carry2):
            sl = pl.ds(c * 16, 16)
            a_v[i, sl] = a_v[i, sl] + b_v[i, sl]
            return carry2

        lax.fori_loop(0, _D // 16, col, 0, unroll=4)
        return carry

    for s in range(_CHUNK // _SUB):
        b2 = base + s * _SUB
        pltpu.sync_copy(pos0.at[pl.ds(b2, _SUB)], idx_v)
        pltpu.async_copy(y.at[idx_v], a_v, sem).wait()
        pltpu.sync_copy(pos1.at[pl.ds(b2, _SUB)], idx_v)
        pltpu.async_copy(y.at[idx_v], b_v, sem).wait()
        lax.fori_loop(0, _SUB, addinto, 0)
        pltpu.sync_copy(x.at[pl.ds(b2, _SUB)], b_v)
        lax.fori_loop(0, _SUB, addinto, 0)
        pltpu.sync_copy(a_v, out.at[pl.ds(b2, _SUB)])


def _combine(x2, y, pos0, pos1):
    k = functools.partial(
        pl.kernel,
        mesh=_sc_mesh(),
        out_type=jax.ShapeDtypeStruct((_T, _D), jnp.float32),
        scratch_types=[
            pltpu.VMEM((_SUB,), jnp.int32),
            pltpu.VMEM((_SUB, _D), jnp.float32),
            pltpu.VMEM((_SUB, _D), jnp.float32),
            pltpu.SemaphoreType.DMA,
        ],
    )(_combine_body)
    return k(x2, y, pos0, pos1)


_HB = 512
_NHC = _H // _HB


def _ffn_body(offs_ref, xs_ref, w1_hbm, w3_hbm, w2_hbm, out_ref,
              w1b, w3b, w2b, w1c, w3c, w2c, sem):
    e = pl.program_id(0)

    def issue(ei, hi, slot):
        hs = pl.ds(hi * _HB, _HB)
        pltpu.make_async_copy(w1_hbm.at[ei, :, hs], w1b.at[slot],
                              sem.at[slot]).start()
        pltpu.make_async_copy(w3_hbm.at[ei, :, hs], w3b.at[slot],
                              sem.at[slot]).start()
        pltpu.make_async_copy(w2_hbm.at[ei, hs, :], w2b.at[slot],
                              sem.at[slot]).start()

    def drain(ei, hi, slot):
        hs = pl.ds(hi * _HB, _HB)
        pltpu.make_async_copy(w1_hbm.at[ei, :, hs], w1b.at[slot],
                              sem.at[slot]).wait()
        pltpu.make_async_copy(w3_hbm.at[ei, :, hs], w3b.at[slot],
                              sem.at[slot]).wait()
        pltpu.make_async_copy(w2_hbm.at[ei, hs, :], w2b.at[slot],
                              sem.at[slot]).wait()

    @pl.when(e == 0)
    def _():
        out_ref[...] = jnp.zeros_like(out_ref)
        issue(0, 0, 0)

    start_e = offs_ref[0, e]
    end_e = offs_ref[0, e + 1]
    t_lo = start_e // _TT
    t_hi = (end_e + _TT - 1) // _TT
    riota0 = lax.broadcasted_iota(jnp.int32, (_TT, 1), 0)

    for h in range(_NHC):
        slot = h % 2
        nslot = (h + 1) % 2
        if h + 1 < _NHC:
            issue(e, h + 1, nslot)
        else:
            @pl.when(e + 1 < _E)
            def _():
                issue(e + 1, 0, nslot)
        drain(e, h, slot)

        @pl.when(start_e < end_e)
        def _():
            w1c[...] = w1b[slot].astype(jnp.bfloat16)
            w3c[...] = w3b[slot].astype(jnp.bfloat16)
            w2c[...] = w2b[slot].astype(jnp.bfloat16)

            def tbody(t, carry):
                rows = pl.ds(t * _TT, _TT)
                x = xs_ref[rows, :]
                riota = t * _TT + riota0
                valid = (riota >= start_e) & (riota < end_e)
                xb = jnp.where(valid, x, 0.0).astype(jnp.bfloat16)
                h1 = jnp.dot(xb, w1c[...],
                             preferred_element_type=jnp.float32)
                h3 = jnp.dot(xb, w3c[...],
                             preferred_element_type=jnp.float32)
                hh = (_silu(h1) * h3).astype(jnp.bfloat16)
                y = jnp.dot(hh, w2c[...],
                            preferred_element_type=jnp.float32)
                out_ref[rows, :] += y
                return carry

            lax.fori_loop(t_lo, t_hi, tbody, 0)


def _grouped_ffn(offs, xs, w1, w3, w2):
    grid_spec = pltpu.PrefetchScalarGridSpec(
        num_scalar_prefetch=1,
        grid=(_E,),
        in_specs=[
            pl.BlockSpec((_R, _D), lambda e, o: (0, 0)),
            pl.BlockSpec(memory_space=pl.ANY),
            pl.BlockSpec(memory_space=pl.ANY),
            pl.BlockSpec(memory_space=pl.ANY),
        ],
        out_specs=pl.BlockSpec((_R, _D), lambda e, o: (0, 0)),
        scratch_shapes=[
            pltpu.VMEM((2, _D, _HB), jnp.float32),
            pltpu.VMEM((2, _D, _HB), jnp.float32),
            pltpu.VMEM((2, _HB, _D), jnp.float32),
            pltpu.VMEM((_D, _HB), jnp.bfloat16),
            pltpu.VMEM((_D, _HB), jnp.bfloat16),
            pltpu.VMEM((_HB, _D), jnp.bfloat16),
            pltpu.SemaphoreType.DMA((2,)),
        ],
    )
    return pl.pallas_call(
        _ffn_body,
        grid_spec=grid_spec,
        out_shape=jax.ShapeDtypeStruct((_R, _D), jnp.float32),
        compiler_params=pltpu.CompilerParams(
            dimension_semantics=("arbitrary",),
        ),
    )(offs, xs, w1, w3, w2)


def kernel(X, Wg, W1, W2, W3):
    x2 = X.reshape(_T, _D)

    xg0, xg1, pos0, pos1, offs = _routing(x2, Wg)
    p0 = pos0.reshape(_R // 2)
    p1 = pos1.reshape(_R // 2)

    xs = _dispatch(xg0, xg1, p0, p1)
    y = _grouped_ffn(offs, xs, W1, W3, W2)
    out = _combine(x2, y, p0, p1)
    return out.reshape(X.shape)


# speedup vs baseline: 1.8138x; 1.0377x over previous
import functools

import jax
import jax.numpy as jnp
from jax import lax
from jax.experimental import pallas as pl
from jax.experimental.pallas import tpu as pltpu
from jax.experimental.pallas import tpu_sc as plsc

_D = 768
_E = 8
_H = 3072
_T = 2048
_S = 2
_R = _T * _S
_TT = 256
_NT = _R // _TT

_NW = 32
_CHUNK = _T // _NW


def _silu(x):
    return x / (1.0 + jnp.exp(-x))


def _routing_body(x_ref, wg_ref, xg0_ref, xg1_ref, pos0_ref, pos1_ref,
                  offs_ref):
    x = x_ref[...]
    logits = jnp.dot(x, wg_ref[...], preferred_element_type=jnp.float32)
    iota = lax.broadcasted_iota(jnp.int32, logits.shape, 1)
    m1 = jnp.max(logits, axis=1, keepdims=True)
    oh1 = logits == m1
    idx1 = jnp.min(jnp.where(oh1, iota, _E), axis=1, keepdims=True)
    oh1 = iota == idx1
    neg = jnp.float32(-3.0e38)
    logits2 = jnp.where(oh1, neg, logits)
    m2 = jnp.max(logits2, axis=1, keepdims=True)
    idx2 = jnp.min(jnp.where(logits2 == m2, iota, _E), axis=1, keepdims=True)
    oh2 = iota == idx2
    z = jnp.exp(m2 - m1)
    g1 = 1.0 / (1.0 + z)
    g2 = z / (1.0 + z)

    m = oh1.astype(jnp.float32) + oh2.astype(jnp.float32)
    r_iota = lax.broadcasted_iota(jnp.int32, (_T, _T), 0)
    c_iota = lax.broadcasted_iota(jnp.int32, (_T, _T), 1)
    tril = (r_iota > c_iota).astype(jnp.float32)
    rank = jnp.dot(tril, m, preferred_element_type=jnp.float32)
    counts = jnp.sum(m, axis=0, keepdims=True)
    c16 = jnp.concatenate([counts, jnp.zeros((1, 16 - _E), jnp.float32)],
                          axis=1)
    s = jnp.concatenate([jnp.zeros((1, 1), jnp.float32), c16[:, :15]],
                        axis=1)
    for k in (1, 2, 4, 8):
        s = s + jnp.concatenate(
            [jnp.zeros((1, k), jnp.float32), s[:, :16 - k]], axis=1)
    offs = s

    off8 = offs[:, :_E]
    o1 = jnp.sum(jnp.where(oh1, off8, 0.0), axis=1, keepdims=True)
    o2 = jnp.sum(jnp.where(oh2, off8, 0.0), axis=1, keepdims=True)
    r1 = jnp.sum(jnp.where(oh1, rank, 0.0), axis=1, keepdims=True)
    r2 = jnp.sum(jnp.where(oh2, rank, 0.0), axis=1, keepdims=True)
    pos0_ref[...] = (o1 + r1).astype(jnp.int32)
    pos1_ref[...] = (o2 + r2).astype(jnp.int32)
    xg0_ref[...] = x * g1
    xg1_ref[...] = x * g2
    offs_ref[...] = offs.astype(jnp.int32)


def _routing(x2, wg):
    return pl.pallas_call(
        _routing_body,
        out_shape=(
            jax.ShapeDtypeStruct((_T, _D), jnp.float32),
            jax.ShapeDtypeStruct((_T, _D), jnp.float32),
            jax.ShapeDtypeStruct((_T, 1), jnp.int32),
            jax.ShapeDtypeStruct((_T, 1), jnp.int32),
            jax.ShapeDtypeStruct((1, 16), jnp.int32),
        ),
    )(x2, wg)


def _sc_mesh():
    return plsc.VectorSubcoreMesh(core_axis_name="c", subcore_axis_name="s")


def _dispatch_body(xg0, xg1, pos0, pos1, xs_out, idx_v, rows_v, sem):
    wid = lax.axis_index("s") * 2 + lax.axis_index("c")
    base = wid * _CHUNK
    pltpu.sync_copy(pos0.at[pl.ds(base, _CHUNK)], idx_v)
    pltpu.sync_copy(xg0.at[pl.ds(base, _CHUNK)], rows_v)
    pltpu.async_copy(rows_v, xs_out.at[idx_v], sem).wait()
    pltpu.sync_copy(pos1.at[pl.ds(base, _CHUNK)], idx_v)
    pltpu.sync_copy(xg1.at[pl.ds(base, _CHUNK)], rows_v)
    pltpu.async_copy(rows_v, xs_out.at[idx_v], sem).wait()


def _dispatch(xg0, xg1, pos0, pos1):
    k = functools.partial(
        pl.kernel,
        mesh=_sc_mesh(),
        out_type=jax.ShapeDtypeStruct((_R, _D), jnp.float32),
        scratch_types=[
            pltpu.VMEM((_CHUNK,), jnp.int32),
            pltpu.VMEM((_CHUNK, _D), jnp.float32),
            pltpu.SemaphoreType.DMA,
        ],
    )(_dispatch_body)
    return k(xg0, xg1, pos0, pos1)


_SUB = 32


def _combine_body(x, y, pos0, pos1, out, idx_v, a_v, b_v, sem):
    wid = lax.axis_index("s") * 2 + lax.axis_index("c")
    base = wid * _CHUNK

    def addinto(i, carry):
        def col(c, carry2):
            sl = pl.ds(c * 16, 16)
            a_v[i, sl] = a_v[i, sl] + b_v[i, sl]
            return carry2

        lax.fori_loop(0, _D // 16, col, 0, unroll=4)
        return carry

    for s in range(_CHUNK // _SUB):
        b2 = base + s * _SUB
        pltpu.sync_copy(pos0.at[pl.ds(b2, _SUB)], idx_v)
        pltpu.async_copy(y.at[idx_v], a_v, sem).wait()
        pltpu.sync_copy(pos1.at[pl.ds(b2, _SUB)], idx_v)
        pltpu.async_copy(y.at[idx_v], b_v, sem).wait()
        lax.fori_loop(0, _SUB, addinto, 0)
        pltpu.sync_copy(x.at[pl.ds(b2, _SUB)], b_v)
        lax.fori_loop(0, _SUB, addinto, 0)
        pltpu.sync_copy(a_v, out.at[pl.ds(b2, _SUB)])


def _combine(x2, y, pos0, pos1):
    k = functools.partial(
        pl.kernel,
        mesh=_sc_mesh(),
        out_type=jax.ShapeDtypeStruct((_T, _D), jnp.float32),
        scratch_types=[
            pltpu.VMEM((_SUB,), jnp.int32),
            pltpu.VMEM((_SUB, _D), jnp.float32),
            pltpu.VMEM((_SUB, _D), jnp.float32),
            pltpu.SemaphoreType.DMA,
        ],
    )(_combine_body)
    return k(x2, y, pos0, pos1)


_HB = 512
_NHC = _H // _HB


def _ffn_body(offs_ref, xs_ref, w1_hbm, w3_hbm, w2_hbm, out_ref,
              w1b, w3b, w2b, w1c, w3c, w2c, sem):
    e = pl.program_id(0)

    def issue(ei, hi, slot):
        hs = pl.ds(hi * _HB, _HB)
        pltpu.make_async_copy(w1_hbm.at[ei, :, hs], w1b.at[slot],
                              sem.at[slot]).start()
        pltpu.make_async_copy(w3_hbm.at[ei, :, hs], w3b.at[slot],
                              sem.at[slot]).start()
        pltpu.make_async_copy(w2_hbm.at[ei, hs, :], w2b.at[slot],
                              sem.at[slot]).start()

    def drain(ei, hi, slot):
        hs = pl.ds(hi * _HB, _HB)
        pltpu.make_async_copy(w1_hbm.at[ei, :, hs], w1b.at[slot],
                              sem.at[slot]).wait()
        pltpu.make_async_copy(w3_hbm.at[ei, :, hs], w3b.at[slot],
                              sem.at[slot]).wait()
        pltpu.make_async_copy(w2_hbm.at[ei, hs, :], w2b.at[slot],
                              sem.at[slot]).wait()

    @pl.when(e == 0)
    def _():
        out_ref[...] = jnp.zeros_like(out_ref)
        issue(0, 0, 0)

    start_e = offs_ref[0, e]
    end_e = offs_ref[0, e + 1]
    t_lo = start_e // _TT
    t_hi = (end_e + _TT - 1) // _TT
    riota0 = lax.broadcasted_iota(jnp.int32, (_TT, 1), 0)

    for h in range(_NHC):
        slot = h % 2
        nslot = (h + 1) % 2
        if h + 1 < _NHC:
            issue(e, h + 1, nslot)
        else:
            @pl.when(e + 1 < _E)
            def _():
                issue(e + 1, 0, nslot)
        drain(e, h, slot)

        @pl.when(start_e < end_e)
        def _():
            def tbody(t, carry):
                rows = pl.ds(t * _TT, _TT)
                x = xs_ref[rows, :]
                riota = t * _TT + riota0
                valid = (riota >= start_e) & (riota < end_e)
                xb = jnp.where(valid, x, 0.0)
                h1 = jnp.dot(xb, w1b[slot],
                             preferred_element_type=jnp.float32)
                h3 = jnp.dot(xb, w3b[slot],
                             preferred_element_type=jnp.float32)
                hh = _silu(h1) * h3
                y = jnp.dot(hh, w2b[slot],
                            preferred_element_type=jnp.float32)
                out_ref[rows, :] += y
                return carry

            lax.fori_loop(t_lo, t_hi, tbody, 0)


def _grouped_ffn(offs, xs, w1, w3, w2):
    grid_spec = pltpu.PrefetchScalarGridSpec(
        num_scalar_prefetch=1,
        grid=(_E,),
        in_specs=[
            pl.BlockSpec((_R, _D), lambda e, o: (0, 0)),
            pl.BlockSpec(memory_space=pl.ANY),
            pl.BlockSpec(memory_space=pl.ANY),
            pl.BlockSpec(memory_space=pl.ANY),
        ],
        out_specs=pl.BlockSpec((_R, _D), lambda e, o: (0, 0)),
        scratch_shapes=[
            pltpu.VMEM((2, _D, _HB), jnp.float32),
            pltpu.VMEM((2, _D, _HB), jnp.float32),
            pltpu.VMEM((2, _HB, _D), jnp.float32),
            pltpu.VMEM((_D, _HB), jnp.bfloat16),
            pltpu.VMEM((_D, _HB), jnp.bfloat16),
            pltpu.VMEM((_HB, _D), jnp.bfloat16),
            pltpu.SemaphoreType.DMA((2,)),
        ],
    )
    return pl.pallas_call(
        _ffn_body,
        grid_spec=grid_spec,
        out_shape=jax.ShapeDtypeStruct((_R, _D), jnp.float32),
        compiler_params=pltpu.CompilerParams(
            dimension_semantics=("arbitrary",),
        ),
    )(offs, xs, w1, w3, w2)


def kernel(X, Wg, W1, W2, W3):
    x2 = X.reshape(_T, _D)

    xg0, xg1, pos0, pos1, offs = _routing(x2, Wg)
    p0 = pos0.reshape(_R // 2)
    p1 = pos1.reshape(_R // 2)

    xs = _dispatch(xg0, xg1, p0, p1)
    y = _grouped_ffn(offs, xs, W1, W3, W2)
    out = _combine(x2, y, p0, p1)
    return out.reshape(X.shape)


# speedup vs baseline: 1.9423x; 1.0709x over previous
import functools

import jax
import jax.numpy as jnp
from jax import lax
from jax.experimental import pallas as pl
from jax.experimental.pallas import tpu as pltpu
from jax.experimental.pallas import tpu_sc as plsc

_D = 768
_E = 8
_H = 3072
_T = 2048
_S = 2
_R = _T * _S
_TT = 256
_NT = _R // _TT

_NW = 32
_CHUNK = _T // _NW


def _silu(x):
    return x / (1.0 + jnp.exp(-x))


def _routing_body(x_ref, wg_ref, xg0_ref, xg1_ref, pos0_ref, pos1_ref,
                  offs_ref):
    x = x_ref[...]
    logits = jnp.dot(x, wg_ref[...], preferred_element_type=jnp.float32)
    iota = lax.broadcasted_iota(jnp.int32, logits.shape, 1)
    m1 = jnp.max(logits, axis=1, keepdims=True)
    oh1 = logits == m1
    idx1 = jnp.min(jnp.where(oh1, iota, _E), axis=1, keepdims=True)
    oh1 = iota == idx1
    neg = jnp.float32(-3.0e38)
    logits2 = jnp.where(oh1, neg, logits)
    m2 = jnp.max(logits2, axis=1, keepdims=True)
    idx2 = jnp.min(jnp.where(logits2 == m2, iota, _E), axis=1, keepdims=True)
    oh2 = iota == idx2
    z = jnp.exp(m2 - m1)
    g1 = 1.0 / (1.0 + z)
    g2 = z / (1.0 + z)

    m = oh1.astype(jnp.float32) + oh2.astype(jnp.float32)
    r_iota = lax.broadcasted_iota(jnp.int32, (_T, _T), 0)
    c_iota = lax.broadcasted_iota(jnp.int32, (_T, _T), 1)
    tril = (r_iota > c_iota).astype(jnp.float32)
    rank = jnp.dot(tril, m, preferred_element_type=jnp.float32)
    counts = jnp.sum(m, axis=0, keepdims=True)
    c16 = jnp.concatenate([counts, jnp.zeros((1, 16 - _E), jnp.float32)],
                          axis=1)
    s = jnp.concatenate([jnp.zeros((1, 1), jnp.float32), c16[:, :15]],
                        axis=1)
    for k in (1, 2, 4, 8):
        s = s + jnp.concatenate(
            [jnp.zeros((1, k), jnp.float32), s[:, :16 - k]], axis=1)
    offs = s

    off8 = offs[:, :_E]
    o1 = jnp.sum(jnp.where(oh1, off8, 0.0), axis=1, keepdims=True)
    o2 = jnp.sum(jnp.where(oh2, off8, 0.0), axis=1, keepdims=True)
    r1 = jnp.sum(jnp.where(oh1, rank, 0.0), axis=1, keepdims=True)
    r2 = jnp.sum(jnp.where(oh2, rank, 0.0), axis=1, keepdims=True)
    pos0_ref[...] = (o1 + r1).astype(jnp.int32)
    pos1_ref[...] = (o2 + r2).astype(jnp.int32)
    xg0_ref[...] = x * g1
    xg1_ref[...] = x * g2
    offs_ref[...] = offs.astype(jnp.int32)


def _routing(x2, wg):
    return pl.pallas_call(
        _routing_body,
        out_shape=(
            jax.ShapeDtypeStruct((_T, _D), jnp.float32),
            jax.ShapeDtypeStruct((_T, _D), jnp.float32),
            jax.ShapeDtypeStruct((_T, 1), jnp.int32),
            jax.ShapeDtypeStruct((_T, 1), jnp.int32),
            jax.ShapeDtypeStruct((1, 16), jnp.int32),
        ),
    )(x2, wg)


def _sc_mesh():
    return plsc.VectorSubcoreMesh(core_axis_name="c", subcore_axis_name="s")


def _dispatch_body(xg0, xg1, pos0, pos1, xs_out, idx0_v, idx1_v,
                   rows0_v, rows1_v, sem_a, sem_b, sem_c, sem_d):
    wid = lax.axis_index("s") * 2 + lax.axis_index("c")
    base = wid * _CHUNK
    sl = pl.ds(base, _CHUNK)
    pltpu.sync_copy(pos0.at[sl], idx0_v)
    pltpu.sync_copy(pos1.at[sl], idx1_v)
    ca = pltpu.async_copy(xg0.at[sl], rows0_v, sem_a)
    cb = pltpu.async_copy(xg1.at[sl], rows1_v, sem_b)
    ca.wait()
    cc = pltpu.async_copy(rows0_v, xs_out.at[idx0_v], sem_c)
    cb.wait()
    cd = pltpu.async_copy(rows1_v, xs_out.at[idx1_v], sem_d)
    cc.wait()
    cd.wait()


def _dispatch(xg0, xg1, pos0, pos1):
    k = functools.partial(
        pl.kernel,
        mesh=_sc_mesh(),
        out_type=jax.ShapeDtypeStruct((_R, _D), jnp.float32),
        scratch_types=[
            pltpu.VMEM((_CHUNK,), jnp.int32),
            pltpu.VMEM((_CHUNK,), jnp.int32),
            pltpu.VMEM((_CHUNK, _D), jnp.float32),
            pltpu.VMEM((_CHUNK, _D), jnp.float32),
            pltpu.SemaphoreType.DMA,
            pltpu.SemaphoreType.DMA,
            pltpu.SemaphoreType.DMA,
            pltpu.SemaphoreType.DMA,
        ],
    )(_dispatch_body)
    return k(xg0, xg1, pos0, pos1)


_SUB = 32


def _combine_body(x, y, pos0, pos1, out, idx0_v, idx1_v, a_v, b_v, x_v,
                  sem_a, sem_b, sem_x):
    wid = lax.axis_index("s") * 2 + lax.axis_index("c")
    base = wid * _CHUNK

    def addrow(i, carry):
        def col(c, carry2):
            sl = pl.ds(c * 16, 16)
            a_v[i, sl] = a_v[i, sl] + b_v[i, sl] + x_v[i, sl]
            return carry2

        lax.fori_loop(0, _D // 16, col, 0, unroll=4)
        return carry

    for s in range(_CHUNK // _SUB):
        b2 = base + s * _SUB
        sl = pl.ds(b2, _SUB)
        pltpu.sync_copy(pos0.at[sl], idx0_v)
        pltpu.sync_copy(pos1.at[sl], idx1_v)
        ca = pltpu.async_copy(y.at[idx0_v], a_v, sem_a)
        cb = pltpu.async_copy(y.at[idx1_v], b_v, sem_b)
        cx = pltpu.async_copy(x.at[sl], x_v, sem_x)
        ca.wait()
        cb.wait()
        cx.wait()
        lax.fori_loop(0, _SUB, addrow, 0)
        pltpu.sync_copy(a_v, out.at[sl])


def _combine(x2, y, pos0, pos1):
    k = functools.partial(
        pl.kernel,
        mesh=_sc_mesh(),
        out_type=jax.ShapeDtypeStruct((_T, _D), jnp.float32),
        scratch_types=[
            pltpu.VMEM((_SUB,), jnp.int32),
            pltpu.VMEM((_SUB,), jnp.int32),
            pltpu.VMEM((_SUB, _D), jnp.float32),
            pltpu.VMEM((_SUB, _D), jnp.float32),
            pltpu.VMEM((_SUB, _D), jnp.float32),
            pltpu.SemaphoreType.DMA,
            pltpu.SemaphoreType.DMA,
            pltpu.SemaphoreType.DMA,
        ],
    )(_combine_body)
    return k(x2, y, pos0, pos1)


_HB = 512
_NHC = _H // _HB


def _ffn_body(offs_ref, xs_ref, w1_hbm, w3_hbm, w2_hbm, out_ref,
              w1b, w3b, w2b, sem):
    e = pl.program_id(0)

    def issue(ei, hi, slot):
        hs = pl.ds(hi * _HB, _HB)
        pltpu.make_async_copy(w1_hbm.at[ei, :, hs], w1b.at[slot],
                              sem.at[slot]).start()
        pltpu.make_async_copy(w3_hbm.at[ei, :, hs], w3b.at[slot],
                              sem.at[slot]).start()
        pltpu.make_async_copy(w2_hbm.at[ei, hs, :], w2b.at[slot],
                              sem.at[slot]).start()

    def drain(ei, hi, slot):
        hs = pl.ds(hi * _HB, _HB)
        pltpu.make_async_copy(w1_hbm.at[ei, :, hs], w1b.at[slot],
                              sem.at[slot]).wait()
        pltpu.make_async_copy(w3_hbm.at[ei, :, hs], w3b.at[slot],
                              sem.at[slot]).wait()
        pltpu.make_async_copy(w2_hbm.at[ei, hs, :], w2b.at[slot],
                              sem.at[slot]).wait()

    @pl.when(e == 0)
    def _():
        out_ref[...] = jnp.zeros_like(out_ref)
        issue(0, 0, 0)

    start_e = offs_ref[0, e]
    end_e = offs_ref[0, e + 1]
    t_lo = start_e // _TT
    t_hi = (end_e + _TT - 1) // _TT
    riota0 = lax.broadcasted_iota(jnp.int32, (_TT, 1), 0)

    for h in range(_NHC):
        slot = h % 2
        nslot = (h + 1) % 2
        if h + 1 < _NHC:
            issue(e, h + 1, nslot)
        else:
            @pl.when(e + 1 < _E)
            def _():
                issue(e + 1, 0, nslot)
        drain(e, h, slot)

        @pl.when(start_e < end_e)
        def _():
            def tbody(t, carry):
                rows = pl.ds(t * _TT, _TT)
                x = xs_ref[rows, :]
                riota = t * _TT + riota0
                valid = (riota >= start_e) & (riota < end_e)
                xb = jnp.where(valid, x, 0.0)
                h1 = jnp.dot(xb, w1b[slot],
                             preferred_element_type=jnp.float32)
                h3 = jnp.dot(xb, w3b[slot],
                             preferred_element_type=jnp.float32)
                hh = _silu(h1) * h3
                y = jnp.dot(hh, w2b[slot],
                            preferred_element_type=jnp.float32)
                out_ref[rows, :] += y
                return carry

            lax.fori_loop(t_lo, t_hi, tbody, 0)


def _grouped_ffn(offs, xs, w1, w3, w2):
    grid_spec = pltpu.PrefetchScalarGridSpec(
        num_scalar_prefetch=1,
        grid=(_E,),
        in_specs=[
            pl.BlockSpec((_R, _D), lambda e, o: (0, 0)),
            pl.BlockSpec(memory_space=pl.ANY),
            pl.BlockSpec(memory_space=pl.ANY),
            pl.BlockSpec(memory_space=pl.ANY),
        ],
        out_specs=pl.BlockSpec((_R, _D), lambda e, o: (0, 0)),
        scratch_shapes=[
            pltpu.VMEM((2, _D, _HB), jnp.float32),
            pltpu.VMEM((2, _D, _HB), jnp.float32),
            pltpu.VMEM((2, _HB, _D), jnp.float32),
            pltpu.SemaphoreType.DMA((2,)),
        ],
    )
    return pl.pallas_call(
        _ffn_body,
        grid_spec=grid_spec,
        out_shape=jax.ShapeDtypeStruct((_R, _D), jnp.float32),
        compiler_params=pltpu.CompilerParams(
            dimension_semantics=("arbitrary",),
        ),
    )(offs, xs, w1, w3, w2)


def kernel(X, Wg, W1, W2, W3):
    x2 = X.reshape(_T, _D)

    xg0, xg1, pos0, pos1, offs = _routing(x2, Wg)
    p0 = pos0.reshape(_R // 2)
    p1 = pos1.reshape(_R // 2)

    xs = _dispatch(xg0, xg1, p0, p1)
    y = _grouped_ffn(offs, xs, W1, W3, W2)
    out = _combine(x2, y, p0, p1)
    return out.reshape(X.shape)
